# Initial kernel scaffold; baseline (speedup 1.0000x reference)
import jax
import jax.numpy as jnp
from jax import lax
from jax.experimental import pallas as pl
from jax.experimental.pallas import tpu as pltpu
from jax.experimental.pallas import tpu_sc as plsc

_N = 100000
_E = 6400000
_ALPHA = 2.0
_NC = 2
_NS = 16
_NW = _NC * _NS
_EPW = _E // _NW
_CHUNK = 2000
_NCHUNKS = _EPW // _CHUNK
_VSTEPS = _CHUNK // 16
_NPAD = 102400
_SL = _NPAD // _NS


def _sc_body(pos4, sender, receiver, shifts, zeros, part,
             sidx, ridx, shf, srows, rrows, ebuf, acc, sem):
    c = lax.axis_index("c")
    s = lax.axis_index("s")
    wid = s * _NC + c

    pltpu.sync_copy(zeros.at[pl.ds(s * _SL, _SL)], acc.at[pl.ds(s * _SL, _SL)])
    plsc.subcore_barrier()

    lanes16 = lax.iota(jnp.int32, 16)
    c0 = jnp.zeros((16,), jnp.int32)

    def chunk_body(g, carry):
        base = pl.multiple_of(wid * _EPW + g * _CHUNK, _CHUNK)
        pltpu.sync_copy(sender.at[pl.ds(base, _CHUNK)], sidx)
        pltpu.sync_copy(receiver.at[pl.ds(base, _CHUNK)], ridx)
        pltpu.sync_copy(shifts.at[pl.ds(pl.multiple_of(base * 3, 8), _CHUNK * 3)], shf)
        d1 = pltpu.async_copy(pos4.at[sidx], srows, sem)
        d2 = pltpu.async_copy(pos4.at[ridx], rrows, sem)
        d1.wait()
        d2.wait()

        def vec_body(i, carry2):
            e = i * 16 + lanes16
            sx = plsc.load_gather(srows, [e, c0])
            sy = plsc.load_gather(srows, [e, c0 + 1])
            sz = plsc.load_gather(srows, [e, c0 + 2])
            rx = plsc.load_gather(rrows, [e, c0])
            ry = plsc.load_gather(rrows, [e, c0 + 1])
            rz = plsc.load_gather(rrows, [e, c0 + 2])
            e3 = e * 3
            hx = plsc.load_gather(shf, [e3])
            hy = plsc.load_gather(shf, [e3 + 1])
            hz = plsc.load_gather(shf, [e3 + 2])
            dx = rx - sx + hx
            dy = ry - sy + hy
            dz = rz - sz + hz
            ssq = dx * dx + dy * dy + dz * dz
            ib = plsc.bitcast(ssq, jnp.int32)
            y = plsc.bitcast(jnp.int32(0x5F3759DF) - lax.shift_right_logical(ib, 1),
                             jnp.float32)
            h = 0.5 * ssq
            y = y * (1.5 - h * y * y)
            y = y * (1.5 - h * y * y)
            y = y * (1.5 - h * y * y)
            ebuf[pl.ds(i * 16, 16)] = jnp.exp((-_ALPHA) * (ssq * y))
            return carry2

        lax.fori_loop(0, _VSTEPS, vec_body, 0, unroll=False)
        pltpu.sync_copy(ebuf, acc.at[ridx], add=True)
        return carry

    lax.fori_loop(0, _NCHUNKS, chunk_body, 0, unroll=False)

    plsc.subcore_barrier()
    off = pl.multiple_of(c * _NPAD + s * _SL, 8)
    pltpu.sync_copy(acc.at[pl.ds(s * _SL, _SL)], part.at[pl.ds(off, _SL)])


def _combine_body(p_ref, o_ref):
    o_ref[...] = 0.5 * (p_ref[0] + p_ref[1])


def kernel(positions, edge_index, shifts):
    pos4 = jnp.pad(positions, ((0, 0), (0, 1)))
    sender = edge_index[0]
    receiver = edge_index[1]
    shifts_flat = shifts.reshape(-1)
    zeros = jnp.zeros((_NPAD,), jnp.float32)
    mesh = plsc.VectorSubcoreMesh(core_axis_name="c", subcore_axis_name="s")
    part = pl.kernel(
        _sc_body,
        out_type=jax.ShapeDtypeStruct((2 * _NPAD,), jnp.float32),
        mesh=mesh,
        scratch_types=[
            pltpu.VMEM((_CHUNK,), jnp.int32),
            pltpu.VMEM((_CHUNK,), jnp.int32),
            pltpu.VMEM((_CHUNK * 3,), jnp.float32),
            pltpu.VMEM((_CHUNK, 4), jnp.float32),
            pltpu.VMEM((_CHUNK, 4), jnp.float32),
            pltpu.VMEM((_CHUNK,), jnp.float32),
            pltpu.VMEM_SHARED((_NPAD,), jnp.float32),
            pltpu.SemaphoreType.DMA,
        ],
    )(pos4, sender, receiver, shifts_flat, zeros)
    p3 = part.reshape(2, 800, 128)
    out_pad = pl.pallas_call(
        _combine_body,
        out_shape=jax.ShapeDtypeStruct((800, 128), jnp.float32),
    )(p3)
    return out_pad.reshape(-1)[:_N]


# speedup vs baseline: 55.3998x; 55.3998x over previous
import jax
import jax.numpy as jnp
from jax import lax
from jax.experimental import pallas as pl
from jax.experimental.pallas import tpu as pltpu
from jax.experimental.pallas import tpu_sc as plsc

_N = 100000
_E = 6400000
_ALPHA = 2.0
_NC = 2
_NS = 16
_NW = _NC * _NS
_EPW = _E // _NW
_CHUNK = 2000
_NCHUNKS = _EPW // _CHUNK
_VSTEPS = _CHUNK // 16
_NPAD = 102400
_SL = _NPAD // _NS


def _sc_body(px, py, pz, sender, receiver, shx, shy, shz, zeros, part,
             sidx, ridx, hx, hy, hz, sxv, syv, szv, rxv, ryv, rzv, ebuf,
             spx, spy, spz, acc, sem):
    c = lax.axis_index("c")
    s = lax.axis_index("s")
    wid = s * _NC + c

    sl = pl.ds(s * _SL, _SL)
    pltpu.sync_copy(px.at[sl], spx.at[sl])
    pltpu.sync_copy(py.at[sl], spy.at[sl])
    pltpu.sync_copy(pz.at[sl], spz.at[sl])
    pltpu.sync_copy(zeros.at[sl], acc.at[sl])
    plsc.subcore_barrier()

    def chunk_body(g, carry):
        base = pl.multiple_of(wid * _EPW + g * _CHUNK, _CHUNK)
        ch = pl.ds(base, _CHUNK)
        pltpu.sync_copy(sender.at[ch], sidx)
        pltpu.sync_copy(receiver.at[ch], ridx)
        pltpu.sync_copy(shx.at[ch], hx)
        pltpu.sync_copy(shy.at[ch], hy)
        pltpu.sync_copy(shz.at[ch], hz)
        d1 = pltpu.async_copy(spx.at[sidx], sxv, sem)
        d2 = pltpu.async_copy(spy.at[sidx], syv, sem)
        d3 = pltpu.async_copy(spz.at[sidx], szv, sem)
        d4 = pltpu.async_copy(spx.at[ridx], rxv, sem)
        d5 = pltpu.async_copy(spy.at[ridx], ryv, sem)
        d6 = pltpu.async_copy(spz.at[ridx], rzv, sem)
        d1.wait(); d2.wait(); d3.wait(); d4.wait(); d5.wait(); d6.wait()

        def vec_body(i, carry2):
            v = pl.ds(i * 16, 16)
            dx = rxv[v] - sxv[v] + hx[v]
            dy = ryv[v] - syv[v] + hy[v]
            dz = rzv[v] - szv[v] + hz[v]
            ssq = dx * dx + dy * dy + dz * dz
            ib = lax.bitcast_convert_type(ssq, jnp.int32)
            y = lax.bitcast_convert_type(
                jnp.int32(0x5F3759DF) - lax.shift_right_logical(ib, 1), jnp.float32)
            h = 0.5 * ssq
            y = y * (1.5 - h * y * y)
            y = y * (1.5 - h * y * y)
            y = y * (1.5 - h * y * y)
            ebuf[v] = jnp.exp((-_ALPHA) * (ssq * y))
            return carry2

        lax.fori_loop(0, _VSTEPS, vec_body, 0, unroll=False)
        pltpu.sync_copy(ebuf, acc.at[ridx], add=True)
        return carry

    lax.fori_loop(0, _NCHUNKS, chunk_body, 0, unroll=False)

    plsc.subcore_barrier()
    off = pl.multiple_of(c * _NPAD + s * _SL, 8)
    pltpu.sync_copy(acc.at[sl], part.at[pl.ds(off, _SL)])


def _combine_body(p_ref, o_ref):
    o_ref[...] = 0.5 * (p_ref[0] + p_ref[1])


def kernel(positions, edge_index, shifts):
    posp = jnp.pad(positions, ((0, _NPAD - _N), (0, 0))).T
    sht = shifts.T
    sender = edge_index[0]
    receiver = edge_index[1]
    zeros = jnp.zeros((_NPAD,), jnp.float32)
    mesh = plsc.VectorSubcoreMesh(core_axis_name="c", subcore_axis_name="s")
    part = pl.kernel(
        _sc_body,
        out_type=jax.ShapeDtypeStruct((2 * _NPAD,), jnp.float32),
        mesh=mesh,
        scratch_types=[
            pltpu.VMEM((_CHUNK,), jnp.int32),
            pltpu.VMEM((_CHUNK,), jnp.int32),
            pltpu.VMEM((_CHUNK,), jnp.float32),
            pltpu.VMEM((_CHUNK,), jnp.float32),
            pltpu.VMEM((_CHUNK,), jnp.float32),
            pltpu.VMEM((_CHUNK,), jnp.float32),
            pltpu.VMEM((_CHUNK,), jnp.float32),
            pltpu.VMEM((_CHUNK,), jnp.float32),
            pltpu.VMEM((_CHUNK,), jnp.float32),
            pltpu.VMEM((_CHUNK,), jnp.float32),
            pltpu.VMEM((_CHUNK,), jnp.float32),
            pltpu.VMEM((_CHUNK,), jnp.float32),
            pltpu.VMEM_SHARED((_NPAD,), jnp.float32),
            pltpu.VMEM_SHARED((_NPAD,), jnp.float32),
            pltpu.VMEM_SHARED((_NPAD,), jnp.float32),
            pltpu.VMEM_SHARED((_NPAD,), jnp.float32),
            pltpu.SemaphoreType.DMA,
        ],
    )(posp[0], posp[1], posp[2], sender, receiver, sht[0], sht[1], sht[2], zeros)
    p3 = part.reshape(2, 800, 128)
    out_pad = pl.pallas_call(
        _combine_body,
        out_shape=jax.ShapeDtypeStruct((800, 128), jnp.float32),
    )(p3)
    return out_pad.reshape(-1)[:_N]


# speedup vs baseline: 79.4377x; 1.4339x over previous
import jax
import jax.numpy as jnp
from jax import lax
from jax.experimental import pallas as pl
from jax.experimental.pallas import tpu as pltpu
from jax.experimental.pallas import tpu_sc as plsc

_N = 100000
_E = 6400000
_ALPHA = 2.0
_NC = 2
_NS = 16
_NW = _NC * _NS
_EPW = _E // _NW
_CHUNK = 4000
_NCHUNKS = _EPW // _CHUNK
_VSTEPS = _CHUNK // 16
_NPAD = 102400
_SL = _NPAD // _NS


def _sc_body(px, py, pz, sender, receiver, shx, shy, shz, zeros, part,
             sidx0, ridx0, hx0, hy0, hz0, sxv0, syv0, szv0, rxv0, ryv0, rzv0, eb0,
             sidx1, ridx1, hx1, hy1, hz1, sxv1, syv1, szv1, rxv1, ryv1, rzv1, eb1,
             spx, spy, spz, acc, semL0, semL1, semG0, semG1):
    c = lax.axis_index("c")
    s = lax.axis_index("s")
    wid = s * _NC + c

    sidx = (sidx0, sidx1)
    ridx = (ridx0, ridx1)
    hx = (hx0, hx1)
    hy = (hy0, hy1)
    hz = (hz0, hz1)
    sxv = (sxv0, sxv1)
    syv = (syv0, syv1)
    szv = (szv0, szv1)
    rxv = (rxv0, rxv1)
    ryv = (ryv0, ryv1)
    rzv = (rzv0, rzv1)
    ebuf = (eb0, eb1)
    semL = (semL0, semL1)
    semG = (semG0, semG1)

    sl = pl.ds(s * _SL, _SL)
    pltpu.sync_copy(px.at[sl], spx.at[sl])
    pltpu.sync_copy(py.at[sl], spy.at[sl])
    pltpu.sync_copy(pz.at[sl], spz.at[sl])
    pltpu.sync_copy(zeros.at[sl], acc.at[sl])
    plsc.subcore_barrier()

    base0 = wid * _EPW

    def lin_start(b, g):
        ch = pl.ds(pl.multiple_of(base0 + g * _CHUNK, _CHUNK), _CHUNK)
        pltpu.async_copy(sender.at[ch], sidx[b], semL[b])
        pltpu.async_copy(receiver.at[ch], ridx[b], semL[b])
        pltpu.async_copy(shx.at[ch], hx[b], semL[b])
        pltpu.async_copy(shy.at[ch], hy[b], semL[b])
        pltpu.async_copy(shz.at[ch], hz[b], semL[b])

    def lin_wait(b):
        ch = pl.ds(0, _CHUNK)
        pltpu.make_async_copy(sender.at[ch], sidx[b], semL[b]).wait()
        pltpu.make_async_copy(receiver.at[ch], ridx[b], semL[b]).wait()
        pltpu.make_async_copy(shx.at[ch], hx[b], semL[b]).wait()
        pltpu.make_async_copy(shy.at[ch], hy[b], semL[b]).wait()
        pltpu.make_async_copy(shz.at[ch], hz[b], semL[b]).wait()

    def gat_start(b):
        pltpu.async_copy(spx.at[sidx[b]], sxv[b], semG[b])
        pltpu.async_copy(spy.at[sidx[b]], syv[b], semG[b])
        pltpu.async_copy(spz.at[sidx[b]], szv[b], semG[b])
        pltpu.async_copy(spx.at[ridx[b]], rxv[b], semG[b])
        pltpu.async_copy(spy.at[ridx[b]], ryv[b], semG[b])
        pltpu.async_copy(spz.at[ridx[b]], rzv[b], semG[b])

    def gat_wait(b):
        pltpu.make_async_copy(spx.at[sidx[b]], sxv[b], semG[b]).wait()
        pltpu.make_async_copy(spy.at[sidx[b]], syv[b], semG[b]).wait()
        pltpu.make_async_copy(spz.at[sidx[b]], szv[b], semG[b]).wait()
        pltpu.make_async_copy(spx.at[ridx[b]], rxv[b], semG[b]).wait()
        pltpu.make_async_copy(spy.at[ridx[b]], ryv[b], semG[b]).wait()
        pltpu.make_async_copy(spz.at[ridx[b]], rzv[b], semG[b]).wait()

    def compute_scatter(b):
        def vec_body(i, carry2):
            v = pl.ds(i * 16, 16)
            dx = rxv[b][v] - sxv[b][v] + hx[b][v]
            dy = ryv[b][v] - syv[b][v] + hy[b][v]
            dz = rzv[b][v] - szv[b][v] + hz[b][v]
            ssq = dx * dx + dy * dy + dz * dz
            ib = lax.bitcast_convert_type(ssq, jnp.int32)
            y = lax.bitcast_convert_type(
                jnp.int32(0x5F3759DF) - lax.shift_right_logical(ib, 1), jnp.float32)
            h = 0.5 * ssq
            y = y * (1.5 - h * y * y)
            y = y * (1.5 - h * y * y)
            ebuf[b][v] = jnp.exp((-_ALPHA) * (ssq * y))
            return carry2

        lax.fori_loop(0, _VSTEPS, vec_body, 0, unroll=4)
        pltpu.sync_copy(ebuf[b], acc.at[ridx[b]], add=True)

    lin_start(0, 0)
    lin_wait(0)
    gat_start(0)
    lin_start(1, 1)

    def pair_body(g2, carry):
        g = g2 * 2
        for b in (0, 1):
            lin_wait(1 - b)
            gat_wait(b)
            gat_start(1 - b)
            compute_scatter(b)
            lin_start(b, g + 2)
            g = g + 1
        return carry

    lax.fori_loop(0, _NCHUNKS // 2 - 1, pair_body, 0, unroll=False)

    lin_wait(1)
    gat_wait(0)
    gat_start(1)
    compute_scatter(0)
    gat_wait(1)
    compute_scatter(1)

    plsc.subcore_barrier()
    off = pl.multiple_of(c * _NPAD + s * _SL, 8)
    pltpu.sync_copy(acc.at[sl], part.at[pl.ds(off, _SL)])


def _combine_body(p_ref, o_ref):
    o_ref[...] = 0.5 * (p_ref[0] + p_ref[1])


def kernel(positions, edge_index, shifts):
    posp = jnp.pad(positions, ((0, _NPAD - _N), (0, 0))).T
    sht = shifts.T
    sender = edge_index[0]
    receiver = edge_index[1]
    zeros = jnp.zeros((_NPAD,), jnp.float32)
    mesh = plsc.VectorSubcoreMesh(core_axis_name="c", subcore_axis_name="s")
    vm_i = lambda: pltpu.VMEM((_CHUNK,), jnp.int32)
    vm_f = lambda: pltpu.VMEM((_CHUNK,), jnp.float32)
    part = pl.kernel(
        _sc_body,
        out_type=jax.ShapeDtypeStruct((2 * _NPAD,), jnp.float32),
        mesh=mesh,
        scratch_types=(
            [vm_i(), vm_i()] + [vm_f() for _ in range(10)]
            + [vm_i(), vm_i()] + [vm_f() for _ in range(10)]
            + [pltpu.VMEM_SHARED((_NPAD,), jnp.float32) for _ in range(4)]
            + [pltpu.SemaphoreType.DMA for _ in range(4)]
        ),
    )(posp[0], posp[1], posp[2], sender, receiver, sht[0], sht[1], sht[2], zeros)
    p3 = part.reshape(2, 800, 128)
    out_pad = pl.pallas_call(
        _combine_body,
        out_shape=jax.ShapeDtypeStruct((800, 128), jnp.float32),
    )(p3)
    return out_pad.reshape(-1)[:_N]


# speedup vs baseline: 85.4751x; 1.0760x over previous
import jax
import jax.numpy as jnp
from jax import lax
from jax.experimental import pallas as pl
from jax.experimental.pallas import tpu as pltpu
from jax.experimental.pallas import tpu_sc as plsc

_N = 100000
_E = 6400000
_ALPHA = 2.0
_NC = 2
_NS = 16
_NW = _NC * _NS
_EPW = _E // _NW
_CHUNK = 4000
_NCHUNKS = _EPW // _CHUNK
_VSTEPS = _CHUNK // 16
_NPAD = 102400
_SL = _NPAD // _NS


def _sc_body(px, py, pz, sender, receiver, shx, shy, shz, zeros, part,
             sidx0, ridx0, hx0, hy0, hz0, sxv0, syv0, szv0, rxv0, ryv0, rzv0, eb0,
             sidx1, ridx1, hx1, hy1, hz1, sxv1, syv1, szv1, rxv1, ryv1, rzv1, eb1,
             spx, spy, spz, acc, semL0, semL1, semG0, semG1):
    c = lax.axis_index("c")
    s = lax.axis_index("s")
    wid = s * _NC + c

    sidx = (sidx0, sidx1)
    ridx = (ridx0, ridx1)
    hx = (hx0, hx1)
    hy = (hy0, hy1)
    hz = (hz0, hz1)
    sxv = (sxv0, sxv1)
    syv = (syv0, syv1)
    szv = (szv0, szv1)
    rxv = (rxv0, rxv1)
    ryv = (ryv0, ryv1)
    rzv = (rzv0, rzv1)
    ebuf = (eb0, eb1)
    semL = (semL0, semL1)
    semG = (semG0, semG1)

    sl = pl.ds(s * _SL, _SL)
    pltpu.sync_copy(px.at[sl], spx.at[sl])
    pltpu.sync_copy(py.at[sl], spy.at[sl])
    pltpu.sync_copy(pz.at[sl], spz.at[sl])
    pltpu.sync_copy(zeros.at[sl], acc.at[sl])
    plsc.subcore_barrier()

    base0 = wid * _EPW

    def lin_start(b, g):
        ch = pl.ds(pl.multiple_of(base0 + g * _CHUNK, _CHUNK), _CHUNK)
        pltpu.async_copy(sender.at[ch], sidx[b], semL[b])
        pltpu.async_copy(receiver.at[ch], ridx[b], semL[b])
        pltpu.async_copy(shx.at[ch], hx[b], semL[b])
        pltpu.async_copy(shy.at[ch], hy[b], semL[b])
        pltpu.async_copy(shz.at[ch], hz[b], semL[b])

    def lin_wait(b):
        ch = pl.ds(0, _CHUNK)
        pltpu.make_async_copy(sender.at[ch], sidx[b], semL[b]).wait()
        pltpu.make_async_copy(receiver.at[ch], ridx[b], semL[b]).wait()
        pltpu.make_async_copy(shx.at[ch], hx[b], semL[b]).wait()
        pltpu.make_async_copy(shy.at[ch], hy[b], semL[b]).wait()
        pltpu.make_async_copy(shz.at[ch], hz[b], semL[b]).wait()

    def gat_start(b):
        pltpu.async_copy(spx.at[sidx[b]], sxv[b], semG[b])
        pltpu.async_copy(spy.at[sidx[b]], syv[b], semG[b])
        pltpu.async_copy(spz.at[sidx[b]], szv[b], semG[b])
        pltpu.async_copy(spx.at[ridx[b]], rxv[b], semG[b])
        pltpu.async_copy(spy.at[ridx[b]], ryv[b], semG[b])
        pltpu.async_copy(spz.at[ridx[b]], rzv[b], semG[b])

    def gat_wait(b):
        pltpu.make_async_copy(spx.at[sidx[b]], sxv[b], semG[b]).wait()
        pltpu.make_async_copy(spy.at[sidx[b]], syv[b], semG[b]).wait()
        pltpu.make_async_copy(spz.at[sidx[b]], szv[b], semG[b]).wait()
        pltpu.make_async_copy(spx.at[ridx[b]], rxv[b], semG[b]).wait()
        pltpu.make_async_copy(spy.at[ridx[b]], ryv[b], semG[b]).wait()
        pltpu.make_async_copy(spz.at[ridx[b]], rzv[b], semG[b]).wait()

    def compute_scatter(b):
        def vec_body(i, carry2):
            v = pl.ds(i * 16, 16)
            dx = rxv[b][v] - sxv[b][v] + hx[b][v]
            dy = ryv[b][v] - syv[b][v] + hy[b][v]
            dz = rzv[b][v] - szv[b][v] + hz[b][v]
            ssq = dx * dx + dy * dy + dz * dz
            ib = lax.bitcast_convert_type(ssq, jnp.int32)
            y = lax.bitcast_convert_type(
                jnp.int32(0x5F3759DF) - lax.shift_right_logical(ib, 1), jnp.float32)
            h = 0.5 * ssq
            y = y * (1.5 - h * y * y)
            y = y * (1.5 - h * y * y)
            ebuf[b][v] = jnp.exp((-_ALPHA) * (ssq * y))
            return carry2

        lax.fori_loop(0, _VSTEPS, vec_body, 0, unroll=False)
        pltpu.sync_copy(ebuf[b], acc.at[ridx[b]], add=True)

    lin_start(0, 0)
    lin_wait(0)
    gat_start(0)
    lin_start(1, 1)

    def pair_body(g2, carry):
        g = g2 * 2
        for b in (0, 1):
            lin_wait(1 - b)
            gat_wait(b)
            gat_start(1 - b)
            compute_scatter(b)
            lin_start(b, g + 2)
            g = g + 1
        return carry

    lax.fori_loop(0, _NCHUNKS // 2 - 1, pair_body, 0, unroll=False)

    lin_wait(1)
    gat_wait(0)
    gat_start(1)
    compute_scatter(0)
    gat_wait(1)
    compute_scatter(1)

    plsc.subcore_barrier()
    off = pl.multiple_of(c * _NPAD + s * _SL, 8)
    pltpu.sync_copy(acc.at[sl], part.at[pl.ds(off, _SL)])


def _combine_body(p_ref, o_ref):
    o_ref[...] = 0.5 * (p_ref[0] + p_ref[1])


def kernel(positions, edge_index, shifts):
    posp = jnp.pad(positions, ((0, _NPAD - _N), (0, 0))).T
    sht = shifts.T
    sender = edge_index[0]
    receiver = edge_index[1]
    zeros = jnp.zeros((_NPAD,), jnp.float32)
    mesh = plsc.VectorSubcoreMesh(core_axis_name="c", subcore_axis_name="s")
    vm_i = lambda: pltpu.VMEM((_CHUNK,), jnp.int32)
    vm_f = lambda: pltpu.VMEM((_CHUNK,), jnp.float32)
    part = pl.kernel(
        _sc_body,
        out_type=jax.ShapeDtypeStruct((2 * _NPAD,), jnp.float32),
        mesh=mesh,
        scratch_types=(
            [vm_i(), vm_i()] + [vm_f() for _ in range(10)]
            + [vm_i(), vm_i()] + [vm_f() for _ in range(10)]
            + [pltpu.VMEM_SHARED((_NPAD,), jnp.float32) for _ in range(4)]
            + [pltpu.SemaphoreType.DMA for _ in range(4)]
        ),
    )(posp[0], posp[1], posp[2], sender, receiver, sht[0], sht[1], sht[2], zeros)
    p3 = part.reshape(2, 800, 128)
    out_pad = pl.pallas_call(
        _combine_body,
        out_shape=jax.ShapeDtypeStruct((800, 128), jnp.float32),
    )(p3)
    return out_pad.reshape(-1)[:_N]


# speedup vs baseline: 86.4655x; 1.0116x over previous
import jax
import jax.numpy as jnp
from jax import lax
from jax.experimental import pallas as pl
from jax.experimental.pallas import tpu as pltpu
from jax.experimental.pallas import tpu_sc as plsc

_N = 100000
_E = 6400000
_ALPHA = 2.0
_NC = 2
_NS = 16
_NW = _NC * _NS
_EPW = _E // _NW
_CHUNK = 4000
_NCHUNKS = _EPW // _CHUNK
_VSTEPS = _CHUNK // 16
_NPAD = 102400
_SL = _NPAD // _NS


def _sc_body(px, py, pz, sender, receiver, shx, shy, shz, zeros, part,
             sidx0, ridx0, hx0, hy0, hz0, sxv0, syv0, szv0, rxv0, ryv0, rzv0, eb0,
             sidx1, ridx1, hx1, hy1, hz1, sxv1, syv1, szv1, rxv1, ryv1, rzv1, eb1,
             spx, spy, spz, acc, semL0, semL1, semG0, semG1):
    c = lax.axis_index("c")
    s = lax.axis_index("s")
    wid = s * _NC + c

    sidx = (sidx0, sidx1)
    ridx = (ridx0, ridx1)
    hx = (hx0, hx1)
    hy = (hy0, hy1)
    hz = (hz0, hz1)
    sxv = (sxv0, sxv1)
    syv = (syv0, syv1)
    szv = (szv0, szv1)
    rxv = (rxv0, rxv1)
    ryv = (ryv0, ryv1)
    rzv = (rzv0, rzv1)
    ebuf = (eb0, eb1)
    semL = (semL0, semL1)
    semG = (semG0, semG1)

    sl = pl.ds(s * _SL, _SL)
    pltpu.sync_copy(px.at[sl], spx.at[sl])
    pltpu.sync_copy(py.at[sl], spy.at[sl])
    pltpu.sync_copy(pz.at[sl], spz.at[sl])
    pltpu.sync_copy(zeros.at[sl], acc.at[sl])
    plsc.subcore_barrier()

    base0 = wid * _EPW

    def lin_start(b, g):
        ch = pl.ds(pl.multiple_of(base0 + g * _CHUNK, _CHUNK), _CHUNK)
        pltpu.async_copy(sender.at[ch], sidx[b], semL[b])
        pltpu.async_copy(receiver.at[ch], ridx[b], semL[b])
        pltpu.async_copy(shx.at[ch], hx[b], semL[b])
        pltpu.async_copy(shy.at[ch], hy[b], semL[b])
        pltpu.async_copy(shz.at[ch], hz[b], semL[b])

    def lin_wait(b):
        ch = pl.ds(0, _CHUNK)
        pltpu.make_async_copy(sender.at[ch], sidx[b], semL[b]).wait()
        pltpu.make_async_copy(receiver.at[ch], ridx[b], semL[b]).wait()
        pltpu.make_async_copy(shx.at[ch], hx[b], semL[b]).wait()
        pltpu.make_async_copy(shy.at[ch], hy[b], semL[b]).wait()
        pltpu.make_async_copy(shz.at[ch], hz[b], semL[b]).wait()

    def gat_start(b):
        pltpu.async_copy(spx.at[sidx[b]], sxv[b], semG[b])
        pltpu.async_copy(spy.at[sidx[b]], syv[b], semG[b])
        pltpu.async_copy(spz.at[sidx[b]], szv[b], semG[b])
        pltpu.async_copy(spx.at[ridx[b]], rxv[b], semG[b])
        pltpu.async_copy(spy.at[ridx[b]], ryv[b], semG[b])
        pltpu.async_copy(spz.at[ridx[b]], rzv[b], semG[b])

    def gat_wait(b):
        pltpu.make_async_copy(spx.at[sidx[b]], sxv[b], semG[b]).wait()
        pltpu.make_async_copy(spy.at[sidx[b]], syv[b], semG[b]).wait()
        pltpu.make_async_copy(spz.at[sidx[b]], szv[b], semG[b]).wait()
        pltpu.make_async_copy(spx.at[ridx[b]], rxv[b], semG[b]).wait()
        pltpu.make_async_copy(spy.at[ridx[b]], ryv[b], semG[b]).wait()
        pltpu.make_async_copy(spz.at[ridx[b]], rzv[b], semG[b]).wait()

    def compute_scatter(b):
        def vec_body(i, carry2):
            v = pl.ds(i * 16, 16)
            dx = rxv[b][v] - sxv[b][v] + hx[b][v]
            dy = ryv[b][v] - syv[b][v] + hy[b][v]
            dz = rzv[b][v] - szv[b][v] + hz[b][v]
            ssq = dx * dx + dy * dy + dz * dz
            ib = lax.bitcast_convert_type(ssq, jnp.int32)
            y = lax.bitcast_convert_type(
                jnp.int32(0x5F3759DF) - lax.shift_right_logical(ib, 1), jnp.float32)
            h = 0.5 * ssq
            y = y * (1.5 - h * y * y)
            y = y * (1.5 - h * y * y)
            y = y * (1.5 - h * y * y)
            ebuf[b][v] = jnp.exp((-_ALPHA) * (ssq * y))
            return carry2

        lax.fori_loop(0, _VSTEPS, vec_body, 0, unroll=False)
        pltpu.sync_copy(ebuf[b], acc.at[ridx[b]], add=True)

    lin_start(0, 0)
    lin_wait(0)
    gat_start(0)
    lin_start(1, 1)

    def pair_body(g2, carry):
        g = g2 * 2
        for b in (0, 1):
            lin_wait(1 - b)
            gat_wait(b)
            gat_start(1 - b)
            compute_scatter(b)
            lin_start(b, g + 2)
            g = g + 1
        return carry

    lax.fori_loop(0, _NCHUNKS // 2 - 1, pair_body, 0, unroll=False)

    lin_wait(1)
    gat_wait(0)
    gat_start(1)
    compute_scatter(0)
    gat_wait(1)
    compute_scatter(1)

    plsc.subcore_barrier()
    off = pl.multiple_of(c * _NPAD + s * _SL, 8)
    pltpu.sync_copy(acc.at[sl], part.at[pl.ds(off, _SL)])


def _combine_body(p_ref, o_ref):
    o_ref[...] = 0.5 * (p_ref[0] + p_ref[1])


def kernel(positions, edge_index, shifts):
    posp = jnp.pad(positions, ((0, _NPAD - _N), (0, 0))).T
    sht = shifts.T
    sender = edge_index[0]
    receiver = edge_index[1]
    zeros = jnp.zeros((_NPAD,), jnp.float32)
    mesh = plsc.VectorSubcoreMesh(core_axis_name="c", subcore_axis_name="s")
    vm_i = lambda: pltpu.VMEM((_CHUNK,), jnp.int32)
    vm_f = lambda: pltpu.VMEM((_CHUNK,), jnp.float32)
    part = pl.kernel(
        _sc_body,
        out_type=jax.ShapeDtypeStruct((2 * _NPAD,), jnp.float32),
        mesh=mesh,
        scratch_types=(
            [vm_i(), vm_i()] + [vm_f() for _ in range(10)]
            + [vm_i(), vm_i()] + [vm_f() for _ in range(10)]
            + [pltpu.VMEM_SHARED((_NPAD,), jnp.float32) for _ in range(4)]
            + [pltpu.SemaphoreType.DMA for _ in range(4)]
        ),
    )(posp[0], posp[1], posp[2], sender, receiver, sht[0], sht[1], sht[2], zeros)
    p3 = part.reshape(2, 800, 128)
    out_pad = pl.pallas_call(
        _combine_body,
        out_shape=jax.ShapeDtypeStruct((800, 128), jnp.float32),
    )(p3)
    return out_pad.reshape(-1)[:_N]
